# baseline (speedup 1.0000x reference)
import functools

import jax
import jax.numpy as jnp
from jax import lax
from jax.experimental import pallas as pl
from jax.experimental.pallas import tpu as pltpu
from jax.experimental.pallas import tpu_sc as plsc

K_TOP = 32
MIX = 0.25
BW = 10.0
NEG = -3.0e38
BIG = 2**30
NG = 512
NCHUNK = 4


def _scores_body(h_ref, k_ref, out_ref):
    kb = k_ref[...]
    s = lax.dot_general(h_ref[...], kb, (((1,), (1,)), ((), ())),
                        preferred_element_type=jnp.float32)
    ksq = jnp.sum(kb * kb, axis=1)
    out_ref[...] = 2.0 * s - ksq[None, :]


def _scores(h, db_keys, bn):
    q, d = h.shape
    n = db_keys.shape[0]
    return pl.pallas_call(
        _scores_body,
        grid=(n // bn,),
        in_specs=[
            pl.BlockSpec((q, d), lambda j: (0, 0)),
            pl.BlockSpec((bn, d), lambda j: (j, 0)),
        ],
        out_specs=pl.BlockSpec((q, bn), lambda j: (0, j)),
        out_shape=jax.ShapeDtypeStruct((q, n), jnp.float32),
    )(h, db_keys)


def _sc_midsection(scores, db_values, vocab):
    q, n = scores.shape
    info = plsc.get_sparse_core_info()
    nc, ns = info.num_cores, info.num_subcores
    nw = nc * ns
    rows_per_w = q // nw
    csz = n // NCHUNK
    tpc = (n // NG) // NCHUNK
    mesh = plsc.VectorSubcoreMesh(core_axis_name="c", subcore_axis_name="s")

    @functools.partial(
        pl.kernel,
        mesh=mesh,
        compiler_params=pltpu.CompilerParams(needs_layout_passes=False),
        out_type=jax.ShapeDtypeStruct((q, vocab), jnp.float32),
        scratch_types=[
            pltpu.VMEM((n,), jnp.float32),
            pltpu.VMEM((NG,), jnp.float32),
            pltpu.VMEM((16,), jnp.float32),
            pltpu.VMEM((K_TOP,), jnp.float32),
            pltpu.VMEM((K_TOP,), jnp.int32),
            pltpu.VMEM((2 * K_TOP,), jnp.int32),
            pltpu.VMEM((vocab,), jnp.float32),
            pltpu.SemaphoreType.DMA,
            pltpu.SemaphoreType.DMA,
            pltpu.SemaphoreType.DMA,
        ],
    )
    def body(scores_hbm, dbv_hbm, out_hbm, row_v, gm_v, gm2_v, tv_v, ti_v,
             tok_v, ebd_v, sem_in, sem_tok, sem_out):
        wid = lax.axis_index("s") * nc + lax.axis_index("c")
        iota = lax.iota(jnp.int32, 16)
        lane0 = iota == 0
        zeros16 = jnp.zeros((16,), jnp.float32)
        negs16 = jnp.full((16,), NEG, jnp.float32)

        def zero_body(i, _):
            ebd_v[pl.ds(i * 16, 16)] = zeros16
            return 0

        lax.fori_loop(0, vocab // 16, zero_body, 0)
        tok_v[pl.ds(0, 16)] = iota * 0
        tok_v[pl.ds(16, 16)] = iota * 0
        tok_v[pl.ds(32, 16)] = iota * 0
        tok_v[pl.ds(48, 16)] = iota * 0

        def do_row(r, _):
            row = wid * rows_per_w + r
            slot = jnp.bitwise_and(r, 1)

            def issue(c, _):
                pltpu.async_copy(
                    scores_hbm.at[row, pl.ds(c * csz, csz)],
                    row_v.at[pl.ds(c * csz, csz)], sem_in)
                return 0

            lax.fori_loop(0, NCHUNK, issue, 0)

            def chunk_body(c, _):
                pltpu.make_async_copy(
                    scores_hbm.at[row, pl.ds(0, csz)],
                    row_v.at[pl.ds(0, csz)], sem_in).wait()
                first = c == 0
                cbase = c * (tpc * NG)
                for v in range(NG // 16):
                    acc = jnp.where(first, negs16, gm_v[pl.ds(v * 16, 16)])
                    for t in range(tpc):
                        acc = jnp.maximum(
                            acc, row_v[pl.ds(cbase + t * NG + v * 16, 16)])
                    gm_v[pl.ds(v * 16, 16)] = acc
                return 0

            lax.fori_loop(0, NCHUNK, chunk_body, 0)

            m2 = negs16
            for j in range(16):
                x = jnp.maximum(gm_v[pl.ds(j * 32, 16)],
                                gm_v[pl.ds(j * 32 + 16, 16)])
                m2 = jnp.where(iota == j, jnp.max(x), m2)
            gm2_v[...] = m2

            def extract(kk, _):
                g2 = gm2_v[...]
                gmax = jnp.max(g2)
                jstar = jnp.min(jnp.where(g2 == gmax, iota, BIG))
                base = jstar * 32
                gva = gm_v[pl.ds(base, 16)]
                gvb = gm_v[pl.ds(base + 16, 16)]
                cand = jnp.minimum(
                    jnp.where(gva == gmax, base + iota, BIG),
                    jnp.where(gvb == gmax, base + 16 + iota, BIG))
                g = jnp.min(cand)

                def probe1(u, carry):
                    pv, m1v, m2v = carry
                    idx_u = g + NG * (u * 16 + iota)
                    val_u = plsc.load_gather(row_v, [idx_u])
                    pv = jnp.minimum(pv,
                                     jnp.where(val_u == gmax, idx_u, BIG))
                    m2v = jnp.maximum(m2v, jnp.minimum(val_u, m1v))
                    m1v = jnp.maximum(m1v, val_u)
                    return pv, m1v, m2v

                pvec, m1v, m2v = lax.fori_loop(
                    0, n // NG // 16, probe1,
                    (jnp.full((16,), BIG, jnp.int32), negs16, negs16))
                estar = jnp.min(pvec)
                estar_v = jnp.full((16,), estar, jnp.int32)
                nm = jnp.max(jnp.where(m1v == gmax, m2v, m1v))

                kk_v = jnp.full((16,), 0, jnp.int32) + kk
                plsc.store_scatter(row_v, [estar_v], negs16, mask=lane0)
                lane = g - base
                ia = jnp.where(lane < 16, lane, 99)
                ib = jnp.where(lane < 16, 99, lane - 16)
                gnew_a = jnp.where(iota == ia, nm, gva)
                gnew_b = jnp.where(iota == ib, nm, gvb)
                gm_v[pl.ds(base, 16)] = gnew_a
                gm_v[pl.ds(base + 16, 16)] = gnew_b
                pmax = jnp.max(jnp.maximum(gnew_a, gnew_b))
                plsc.store_scatter(gm2_v, [jnp.full((16,), jstar, jnp.int32)],
                                   jnp.full((16,), pmax, jnp.float32),
                                   mask=lane0)
                plsc.store_scatter(tv_v, [kk_v],
                                   jnp.full((16,), gmax, jnp.float32),
                                   mask=lane0)
                plsc.store_scatter(ti_v, [kk_v], estar_v, mask=lane0)
                return 0

            lax.fori_loop(0, K_TOP, extract, 0)

            @pl.when(r >= 1)
            def _():
                pltpu.make_async_copy(out_hbm.at[row], ebd_v, sem_out).wait()

            sprev = 1 - slot
            old0 = tok_v[pl.ds(sprev * K_TOP, 16)]
            old1 = tok_v[pl.ds(sprev * K_TOP + 16, 16)]
            plsc.store_scatter(ebd_v, [old0], zeros16)
            plsc.store_scatter(ebd_v, [old1], zeros16)

            tokcp = pltpu.async_copy(
                dbv_hbm.at[ti_v], tok_v.at[pl.ds(slot * K_TOP, K_TOP)],
                sem_tok)

            tv0 = tv_v[pl.ds(0, 16)]
            tv1 = tv_v[pl.ds(16, 16)]
            mx = jnp.max(jnp.maximum(tv0, tv1))
            e0 = jnp.exp((tv0 - mx) / BW)
            e1 = jnp.exp((tv1 - mx) / BW)
            scale = MIX / (zeros16 + jnp.sum(e0 + e1))
            w0 = e0 * scale
            w1 = e1 * scale

            tokcp.wait()
            t0 = tok_v[pl.ds(slot * K_TOP, 16)]
            t1 = tok_v[pl.ds(slot * K_TOP + 16, 16)]

            for j in range(16):
                mj = iota == j
                plsc.addupdate_scatter(ebd_v, [t0], w0, mask=mj)
                plsc.addupdate_scatter(ebd_v, [t1], w1, mask=mj)

            pltpu.async_copy(ebd_v, out_hbm.at[row], sem_out)
            return 0

        lax.fori_loop(0, rows_per_w, do_row, 0)

        pltpu.make_async_copy(out_hbm.at[0], ebd_v, sem_out).wait()

    return body(scores, db_values)


def _mix_body(lg_ref, ebd_ref, out_ref):
    lg = lg_ref[...]
    m = jnp.max(lg, axis=-1, keepdims=True)
    e = jnp.exp(lg - m)
    p = e / jnp.sum(e, axis=-1, keepdims=True)
    out_ref[...] = jnp.log((1.0 - MIX) * p + ebd_ref[...])


def _mix(lg, ebd, br):
    q, v = lg.shape
    return pl.pallas_call(
        _mix_body,
        grid=(q // br,),
        in_specs=[
            pl.BlockSpec((br, v), lambda i: (i, 0)),
            pl.BlockSpec((br, v), lambda i: (i, 0)),
        ],
        out_specs=pl.BlockSpec((br, v), lambda i: (i, 0)),
        out_shape=jax.ShapeDtypeStruct((q, v), jnp.float32),
    )(lg, ebd)


def kernel(hidden, logits, db_keys, db_values):
    b, s_len, d = hidden.shape
    vocab = logits.shape[-1]
    q = b * s_len
    h = hidden.reshape(q, d)
    lg = logits.reshape(q, vocab)

    scores = _scores(h, db_keys, 2048)
    ebd = _sc_midsection(scores, db_values.astype(jnp.int32), vocab)
    out = _mix(lg, ebd, 16)
    return out.reshape(b, s_len, vocab)


# speedup vs baseline: 1.0619x; 1.0619x over previous
import functools

import jax
import jax.numpy as jnp
from jax import lax
from jax.experimental import pallas as pl
from jax.experimental.pallas import tpu as pltpu
from jax.experimental.pallas import tpu_sc as plsc

K_TOP = 32
MIX = 0.25
BW = 10.0
NEG = -3.0e38
BIG = 2**30
NG = 512
NCHUNK = 8


def _scores_body(h_ref, k_ref, out_ref):
    kb = k_ref[...]
    s = lax.dot_general(h_ref[...], kb, (((1,), (1,)), ((), ())),
                        preferred_element_type=jnp.float32)
    ksq = jnp.sum(kb * kb, axis=1)
    out_ref[...] = 2.0 * s - ksq[None, :]


def _scores(h, db_keys, bn):
    q, d = h.shape
    n = db_keys.shape[0]
    return pl.pallas_call(
        _scores_body,
        grid=(n // bn,),
        in_specs=[
            pl.BlockSpec((q, d), lambda j: (0, 0)),
            pl.BlockSpec((bn, d), lambda j: (j, 0)),
        ],
        out_specs=pl.BlockSpec((q, bn), lambda j: (0, j)),
        out_shape=jax.ShapeDtypeStruct((q, n), jnp.float32),
    )(h, db_keys)


def _sc_midsection(scores, db_values, vocab):
    q, n = scores.shape
    info = plsc.get_sparse_core_info()
    nc, ns = info.num_cores, info.num_subcores
    nw = nc * ns
    rows_per_w = q // nw
    csz = n // NCHUNK
    tpc = (n // NG) // NCHUNK
    mesh = plsc.VectorSubcoreMesh(core_axis_name="c", subcore_axis_name="s")

    @functools.partial(
        pl.kernel,
        mesh=mesh,
        compiler_params=pltpu.CompilerParams(needs_layout_passes=False),
        out_type=jax.ShapeDtypeStruct((q, vocab), jnp.float32),
        scratch_types=[
            pltpu.VMEM((n,), jnp.float32),
            pltpu.VMEM((NG,), jnp.float32),
            pltpu.VMEM((16,), jnp.float32),
            pltpu.VMEM((K_TOP,), jnp.float32),
            pltpu.VMEM((K_TOP,), jnp.int32),
            pltpu.VMEM((2 * K_TOP,), jnp.int32),
            pltpu.VMEM((vocab,), jnp.float32),
            pltpu.SemaphoreType.DMA,
            pltpu.SemaphoreType.DMA,
            pltpu.SemaphoreType.DMA,
        ],
    )
    def body(scores_hbm, dbv_hbm, out_hbm, row_v, gm_v, gm2_v, tv_v, ti_v,
             tok_v, ebd_v, sem_in, sem_tok, sem_out):
        wid = lax.axis_index("s") * nc + lax.axis_index("c")
        iota = lax.iota(jnp.int32, 16)
        lane0 = iota == 0
        zeros16 = jnp.zeros((16,), jnp.float32)
        negs16 = jnp.full((16,), NEG, jnp.float32)

        def zero_body(i, _):
            ebd_v[pl.ds(i * 16, 16)] = zeros16
            return 0

        lax.fori_loop(0, vocab // 16, zero_body, 0)
        tok_v[pl.ds(0, 16)] = iota * 0
        tok_v[pl.ds(16, 16)] = iota * 0
        tok_v[pl.ds(32, 16)] = iota * 0
        tok_v[pl.ds(48, 16)] = iota * 0

        def do_row(r, _):
            row = wid * rows_per_w + r
            slot = jnp.bitwise_and(r, 1)

            def issue(c, _):
                pltpu.async_copy(
                    scores_hbm.at[row, pl.ds(c * csz, csz)],
                    row_v.at[pl.ds(c * csz, csz)], sem_in)
                return 0

            lax.fori_loop(0, NCHUNK, issue, 0)

            def chunk_body(c, _):
                pltpu.make_async_copy(
                    scores_hbm.at[row, pl.ds(0, csz)],
                    row_v.at[pl.ds(0, csz)], sem_in).wait()
                first = c == 0
                cbase = c * (tpc * NG)
                for v in range(NG // 16):
                    acc = jnp.where(first, negs16, gm_v[pl.ds(v * 16, 16)])
                    for t in range(tpc):
                        acc = jnp.maximum(
                            acc, row_v[pl.ds(cbase + t * NG + v * 16, 16)])
                    gm_v[pl.ds(v * 16, 16)] = acc
                return 0

            lax.fori_loop(0, NCHUNK, chunk_body, 0)

            m2 = negs16
            for j in range(16):
                x = jnp.maximum(gm_v[pl.ds(j * 32, 16)],
                                gm_v[pl.ds(j * 32 + 16, 16)])
                m2 = jnp.where(iota == j, jnp.max(x), m2)
            gm2_v[...] = m2

            def extract(kk, _):
                g2 = gm2_v[...]
                gmax = jnp.max(g2)
                jstar = jnp.min(jnp.where(g2 == gmax, iota, BIG))
                base = jstar * 32
                gva = gm_v[pl.ds(base, 16)]
                gvb = gm_v[pl.ds(base + 16, 16)]
                cand = jnp.minimum(
                    jnp.where(gva == gmax, base + iota, BIG),
                    jnp.where(gvb == gmax, base + 16 + iota, BIG))
                g = jnp.min(cand)

                def probe1(u, carry):
                    pv, m1v, m2v = carry
                    idx_u = g + NG * (u * 16 + iota)
                    val_u = plsc.load_gather(row_v, [idx_u])
                    pv = jnp.minimum(pv,
                                     jnp.where(val_u == gmax, idx_u, BIG))
                    m2v = jnp.maximum(m2v, jnp.minimum(val_u, m1v))
                    m1v = jnp.maximum(m1v, val_u)
                    return pv, m1v, m2v

                pvec, m1v, m2v = lax.fori_loop(
                    0, n // NG // 16, probe1,
                    (jnp.full((16,), BIG, jnp.int32), negs16, negs16))
                estar = jnp.min(pvec)
                estar_v = jnp.full((16,), estar, jnp.int32)
                elane = jnp.bitwise_and((estar - g) // NG, 15)
                nm = jnp.max(jnp.where(iota == elane, m2v, m1v))

                kk_v = jnp.full((16,), 0, jnp.int32) + kk
                plsc.store_scatter(row_v, [estar_v], negs16, mask=lane0)
                lane = g - base
                ia = jnp.where(lane < 16, lane, 99)
                ib = jnp.where(lane < 16, 99, lane - 16)
                gnew_a = jnp.where(iota == ia, nm, gva)
                gnew_b = jnp.where(iota == ib, nm, gvb)
                gm_v[pl.ds(base, 16)] = gnew_a
                gm_v[pl.ds(base + 16, 16)] = gnew_b
                pmax = jnp.max(jnp.maximum(gnew_a, gnew_b))
                plsc.store_scatter(gm2_v, [jnp.full((16,), jstar, jnp.int32)],
                                   jnp.full((16,), pmax, jnp.float32),
                                   mask=lane0)
                plsc.store_scatter(tv_v, [kk_v],
                                   jnp.full((16,), gmax, jnp.float32),
                                   mask=lane0)
                plsc.store_scatter(ti_v, [kk_v], estar_v, mask=lane0)
                return 0

            lax.fori_loop(0, K_TOP, extract, 0)

            @pl.when(r >= 1)
            def _():
                pltpu.make_async_copy(out_hbm.at[row], ebd_v, sem_out).wait()

            sprev = 1 - slot
            old0 = tok_v[pl.ds(sprev * K_TOP, 16)]
            old1 = tok_v[pl.ds(sprev * K_TOP + 16, 16)]
            plsc.store_scatter(ebd_v, [old0], zeros16)
            plsc.store_scatter(ebd_v, [old1], zeros16)

            tokcp = pltpu.async_copy(
                dbv_hbm.at[ti_v], tok_v.at[pl.ds(slot * K_TOP, K_TOP)],
                sem_tok)

            tv0 = tv_v[pl.ds(0, 16)]
            tv1 = tv_v[pl.ds(16, 16)]
            mx = jnp.max(jnp.maximum(tv0, tv1))
            e0 = jnp.exp((tv0 - mx) / BW)
            e1 = jnp.exp((tv1 - mx) / BW)
            scale = MIX / (zeros16 + jnp.sum(e0 + e1))
            w0 = e0 * scale
            w1 = e1 * scale

            tokcp.wait()
            t0 = tok_v[pl.ds(slot * K_TOP, 16)]
            t1 = tok_v[pl.ds(slot * K_TOP + 16, 16)]

            for j in range(16):
                mj = iota == j
                plsc.addupdate_scatter(ebd_v, [t0], w0, mask=mj)
                plsc.addupdate_scatter(ebd_v, [t1], w1, mask=mj)

            pltpu.async_copy(ebd_v, out_hbm.at[row], sem_out)
            return 0

        lax.fori_loop(0, rows_per_w, do_row, 0)

        pltpu.make_async_copy(out_hbm.at[0], ebd_v, sem_out).wait()

    return body(scores, db_values)


def _mix_body(lg_ref, ebd_ref, out_ref):
    lg = lg_ref[...]
    m = jnp.max(lg, axis=-1, keepdims=True)
    e = jnp.exp(lg - m)
    p = e / jnp.sum(e, axis=-1, keepdims=True)
    out_ref[...] = jnp.log((1.0 - MIX) * p + ebd_ref[...])


def _mix(lg, ebd, br):
    q, v = lg.shape
    return pl.pallas_call(
        _mix_body,
        grid=(q // br,),
        in_specs=[
            pl.BlockSpec((br, v), lambda i: (i, 0)),
            pl.BlockSpec((br, v), lambda i: (i, 0)),
        ],
        out_specs=pl.BlockSpec((br, v), lambda i: (i, 0)),
        out_shape=jax.ShapeDtypeStruct((q, v), jnp.float32),
    )(lg, ebd)


def kernel(hidden, logits, db_keys, db_values):
    b, s_len, d = hidden.shape
    vocab = logits.shape[-1]
    q = b * s_len
    h = hidden.reshape(q, d)
    lg = logits.reshape(q, vocab)

    scores = _scores(h, db_keys, 2048)
    ebd = _sc_midsection(scores, db_values.astype(jnp.int32), vocab)
    out = _mix(lg, ebd, 16)
    return out.reshape(b, s_len, vocab)


# speedup vs baseline: 1.0915x; 1.0278x over previous
import functools

import jax
import jax.numpy as jnp
from jax import lax
from jax.experimental import pallas as pl
from jax.experimental.pallas import tpu as pltpu
from jax.experimental.pallas import tpu_sc as plsc

K_TOP = 32
MIX = 0.25
BW = 10.0
NEG = -3.0e38
BIG = 2**30
NG = 512
NCHUNK = 8


def _scores_body(h_ref, k_ref, out_ref):
    kb = k_ref[...]
    s = lax.dot_general(h_ref[...], kb, (((1,), (1,)), ((), ())),
                        preferred_element_type=jnp.float32)
    ksq = jnp.sum(kb * kb, axis=1)
    out_ref[...] = 2.0 * s - ksq[None, :]


def _scores(h, db_keys, bn):
    q, d = h.shape
    n = db_keys.shape[0]
    return pl.pallas_call(
        _scores_body,
        grid=(n // bn,),
        in_specs=[
            pl.BlockSpec((q, d), lambda j: (0, 0)),
            pl.BlockSpec((bn, d), lambda j: (j, 0)),
        ],
        out_specs=pl.BlockSpec((q, bn), lambda j: (0, j)),
        out_shape=jax.ShapeDtypeStruct((q, n), jnp.float32),
    )(h, db_keys)


def _sc_midsection(scores, db_values, vocab):
    q, n = scores.shape
    info = plsc.get_sparse_core_info()
    nc, ns = info.num_cores, info.num_subcores
    nw = nc * ns
    rows_per_w = q // nw
    csz = n // NCHUNK
    tpc = (n // NG) // NCHUNK
    mesh = plsc.VectorSubcoreMesh(core_axis_name="c", subcore_axis_name="s")

    @functools.partial(
        pl.kernel,
        mesh=mesh,
        compiler_params=pltpu.CompilerParams(needs_layout_passes=False),
        out_type=jax.ShapeDtypeStruct((q, vocab), jnp.float32),
        scratch_types=[
            pltpu.VMEM((n,), jnp.float32),
            pltpu.VMEM((NG,), jnp.float32),
            pltpu.VMEM((16,), jnp.float32),
            pltpu.VMEM((K_TOP,), jnp.float32),
            pltpu.VMEM((K_TOP,), jnp.int32),
            pltpu.VMEM((2 * K_TOP,), jnp.int32),
            pltpu.VMEM((vocab,), jnp.float32),
            pltpu.SemaphoreType.DMA,
            pltpu.SemaphoreType.DMA,
            pltpu.SemaphoreType.DMA,
        ],
    )
    def body(scores_hbm, dbv_hbm, out_hbm, row_v, gm_v, gm2_v, tv_v, ti_v,
             tok_v, ebd_v, sem_in, sem_tok, sem_out):
        wid = lax.axis_index("s") * nc + lax.axis_index("c")
        iota = lax.iota(jnp.int32, 16)
        lane0 = iota == 0
        zeros16 = jnp.zeros((16,), jnp.float32)
        negs16 = jnp.full((16,), NEG, jnp.float32)

        def zero_body(i, _):
            ebd_v[pl.ds(i * 16, 16)] = zeros16
            return 0

        lax.fori_loop(0, vocab // 16, zero_body, 0)
        tok_v[pl.ds(0, 16)] = iota * 0
        tok_v[pl.ds(16, 16)] = iota * 0
        tok_v[pl.ds(32, 16)] = iota * 0
        tok_v[pl.ds(48, 16)] = iota * 0

        def issue_row(row):
            def issue(c, _):
                pltpu.async_copy(
                    scores_hbm.at[row, pl.ds(c * csz, csz)],
                    row_v.at[pl.ds(c * csz, csz)], sem_in)
                return 0

            lax.fori_loop(0, NCHUNK, issue, 0)

        issue_row(wid * rows_per_w)

        def do_row(r, _):
            row = wid * rows_per_w + r
            slot = jnp.bitwise_and(r, 1)

            def chunk_body(c, _):
                pltpu.make_async_copy(
                    scores_hbm.at[row, pl.ds(0, csz)],
                    row_v.at[pl.ds(0, csz)], sem_in).wait()
                first = c == 0
                cbase = c * (tpc * NG)
                for v in range(NG // 16):
                    acc = jnp.where(first, negs16, gm_v[pl.ds(v * 16, 16)])
                    for t in range(tpc):
                        acc = jnp.maximum(
                            acc, row_v[pl.ds(cbase + t * NG + v * 16, 16)])
                    gm_v[pl.ds(v * 16, 16)] = acc
                return 0

            lax.fori_loop(0, NCHUNK, chunk_body, 0)

            m2 = negs16
            for j in range(16):
                x = jnp.maximum(gm_v[pl.ds(j * 32, 16)],
                                gm_v[pl.ds(j * 32 + 16, 16)])
                m2 = jnp.where(iota == j, jnp.max(x), m2)
            gm2_v[...] = m2

            def extract(kk, _):
                g2 = gm2_v[...]
                gmax = jnp.max(g2)
                jstar = jnp.min(jnp.where(g2 == gmax, iota, BIG))
                base = jstar * 32
                gva = gm_v[pl.ds(base, 16)]
                gvb = gm_v[pl.ds(base + 16, 16)]
                cand = jnp.minimum(
                    jnp.where(gva == gmax, base + iota, BIG),
                    jnp.where(gvb == gmax, base + 16 + iota, BIG))
                g = jnp.min(cand)

                def probe1(u, carry):
                    pv, m1v, m2v = carry
                    idx_u = g + NG * (u * 16 + iota)
                    val_u = plsc.load_gather(row_v, [idx_u])
                    pv = jnp.minimum(pv,
                                     jnp.where(val_u == gmax, idx_u, BIG))
                    m2v = jnp.maximum(m2v, jnp.minimum(val_u, m1v))
                    m1v = jnp.maximum(m1v, val_u)
                    return pv, m1v, m2v

                pvec, m1v, m2v = lax.fori_loop(
                    0, n // NG // 16, probe1,
                    (jnp.full((16,), BIG, jnp.int32), negs16, negs16))
                estar = jnp.min(pvec)
                estar_v = jnp.full((16,), estar, jnp.int32)
                elane = jnp.bitwise_and((estar - g) // NG, 15)
                nm = jnp.max(jnp.where(iota == elane, m2v, m1v))

                kk_v = jnp.full((16,), 0, jnp.int32) + kk
                plsc.store_scatter(row_v, [estar_v], negs16, mask=lane0)
                lane = g - base
                ia = jnp.where(lane < 16, lane, 99)
                ib = jnp.where(lane < 16, 99, lane - 16)
                gnew_a = jnp.where(iota == ia, nm, gva)
                gnew_b = jnp.where(iota == ib, nm, gvb)
                gm_v[pl.ds(base, 16)] = gnew_a
                gm_v[pl.ds(base + 16, 16)] = gnew_b
                pmax = jnp.max(jnp.maximum(gnew_a, gnew_b))
                plsc.store_scatter(gm2_v, [jnp.full((16,), jstar, jnp.int32)],
                                   jnp.full((16,), pmax, jnp.float32),
                                   mask=lane0)
                plsc.store_scatter(tv_v, [kk_v],
                                   jnp.full((16,), gmax, jnp.float32),
                                   mask=lane0)
                plsc.store_scatter(ti_v, [kk_v], estar_v, mask=lane0)
                return 0

            lax.fori_loop(0, K_TOP, extract, 0)

            @pl.when(r + 1 < rows_per_w)
            def _():
                issue_row(row + 1)

            @pl.when(r >= 1)
            def _():
                pltpu.make_async_copy(out_hbm.at[row], ebd_v, sem_out).wait()

            sprev = 1 - slot
            old0 = tok_v[pl.ds(sprev * K_TOP, 16)]
            old1 = tok_v[pl.ds(sprev * K_TOP + 16, 16)]
            plsc.store_scatter(ebd_v, [old0], zeros16)
            plsc.store_scatter(ebd_v, [old1], zeros16)

            tokcp = pltpu.async_copy(
                dbv_hbm.at[ti_v], tok_v.at[pl.ds(slot * K_TOP, K_TOP)],
                sem_tok)

            tv0 = tv_v[pl.ds(0, 16)]
            tv1 = tv_v[pl.ds(16, 16)]
            mx = jnp.max(jnp.maximum(tv0, tv1))
            e0 = jnp.exp((tv0 - mx) / BW)
            e1 = jnp.exp((tv1 - mx) / BW)
            scale = MIX / (zeros16 + jnp.sum(e0 + e1))
            w0 = e0 * scale
            w1 = e1 * scale

            tokcp.wait()
            t0 = tok_v[pl.ds(slot * K_TOP, 16)]
            t1 = tok_v[pl.ds(slot * K_TOP + 16, 16)]

            for j in range(16):
                mj = iota == j
                plsc.addupdate_scatter(ebd_v, [t0], w0, mask=mj)
                plsc.addupdate_scatter(ebd_v, [t1], w1, mask=mj)

            pltpu.async_copy(ebd_v, out_hbm.at[row], sem_out)
            return 0

        lax.fori_loop(0, rows_per_w, do_row, 0)

        pltpu.make_async_copy(out_hbm.at[0], ebd_v, sem_out).wait()

    return body(scores, db_values)


def _mix_body(lg_ref, ebd_ref, out_ref):
    lg = lg_ref[...]
    m = jnp.max(lg, axis=-1, keepdims=True)
    e = jnp.exp(lg - m)
    p = e / jnp.sum(e, axis=-1, keepdims=True)
    out_ref[...] = jnp.log((1.0 - MIX) * p + ebd_ref[...])


def _mix(lg, ebd, br):
    q, v = lg.shape
    return pl.pallas_call(
        _mix_body,
        grid=(q // br,),
        in_specs=[
            pl.BlockSpec((br, v), lambda i: (i, 0)),
            pl.BlockSpec((br, v), lambda i: (i, 0)),
        ],
        out_specs=pl.BlockSpec((br, v), lambda i: (i, 0)),
        out_shape=jax.ShapeDtypeStruct((q, v), jnp.float32),
    )(lg, ebd)


def kernel(hidden, logits, db_keys, db_values):
    b, s_len, d = hidden.shape
    vocab = logits.shape[-1]
    q = b * s_len
    h = hidden.reshape(q, d)
    lg = logits.reshape(q, vocab)

    scores = _scores(h, db_keys, 2048)
    ebd = _sc_midsection(scores, db_values.astype(jnp.int32), vocab)
    out = _mix(lg, ebd, 16)
    return out.reshape(b, s_len, vocab)


# speedup vs baseline: 1.1154x; 1.0219x over previous
import functools

import jax
import jax.numpy as jnp
from jax import lax
from jax.experimental import pallas as pl
from jax.experimental.pallas import tpu as pltpu
from jax.experimental.pallas import tpu_sc as plsc

K_TOP = 32
MIX = 0.25
BW = 10.0
NEG = -3.0e38
BIG = 2**30
NG = 512
NCHUNK = 16


def _scores_body(h_ref, k_ref, out_ref):
    kb = k_ref[...]
    s = lax.dot_general(h_ref[...], kb, (((1,), (1,)), ((), ())),
                        preferred_element_type=jnp.float32)
    ksq = jnp.sum(kb * kb, axis=1)
    out_ref[...] = 2.0 * s - ksq[None, :]


def _scores(h, db_keys, bn):
    q, d = h.shape
    n = db_keys.shape[0]
    return pl.pallas_call(
        _scores_body,
        grid=(n // bn,),
        in_specs=[
            pl.BlockSpec((q, d), lambda j: (0, 0)),
            pl.BlockSpec((bn, d), lambda j: (j, 0)),
        ],
        out_specs=pl.BlockSpec((q, bn), lambda j: (0, j)),
        out_shape=jax.ShapeDtypeStruct((q, n), jnp.float32),
    )(h, db_keys)


def _sc_midsection(scores, db_values, vocab):
    q, n = scores.shape
    info = plsc.get_sparse_core_info()
    nc, ns = info.num_cores, info.num_subcores
    nw = nc * ns
    rows_per_w = q // nw
    csz = n // NCHUNK
    tpc = (n // NG) // NCHUNK
    mesh = plsc.VectorSubcoreMesh(core_axis_name="c", subcore_axis_name="s")

    @functools.partial(
        pl.kernel,
        mesh=mesh,
        compiler_params=pltpu.CompilerParams(needs_layout_passes=False),
        out_type=jax.ShapeDtypeStruct((q, vocab), jnp.float32),
        scratch_types=[
            pltpu.VMEM((n,), jnp.float32),
            pltpu.VMEM((NG,), jnp.float32),
            pltpu.VMEM((16,), jnp.float32),
            pltpu.VMEM((K_TOP,), jnp.float32),
            pltpu.VMEM((K_TOP,), jnp.int32),
            pltpu.VMEM((2 * K_TOP,), jnp.int32),
            pltpu.VMEM((vocab,), jnp.float32),
            pltpu.SemaphoreType.DMA,
            pltpu.SemaphoreType.DMA,
            pltpu.SemaphoreType.DMA,
        ],
    )
    def body(scores_hbm, dbv_hbm, out_hbm, row_v, gm_v, gm2_v, tv_v, ti_v,
             tok_v, ebd_v, sem_in, sem_tok, sem_out):
        wid = lax.axis_index("s") * nc + lax.axis_index("c")
        iota = lax.iota(jnp.int32, 16)
        lane0 = iota == 0
        zeros16 = jnp.zeros((16,), jnp.float32)
        negs16 = jnp.full((16,), NEG, jnp.float32)

        def zero_body(i, _):
            ebd_v[pl.ds(i * 16, 16)] = zeros16
            return 0

        lax.fori_loop(0, vocab // 16, zero_body, 0)
        tok_v[pl.ds(0, 16)] = iota * 0
        tok_v[pl.ds(16, 16)] = iota * 0
        tok_v[pl.ds(32, 16)] = iota * 0
        tok_v[pl.ds(48, 16)] = iota * 0

        def issue_row(row):
            def issue(c, _):
                pltpu.async_copy(
                    scores_hbm.at[row, pl.ds(c * csz, csz)],
                    row_v.at[pl.ds(c * csz, csz)], sem_in)
                return 0

            lax.fori_loop(0, NCHUNK, issue, 0)

        issue_row(wid * rows_per_w)

        def do_row(r, _):
            row = wid * rows_per_w + r
            slot = jnp.bitwise_and(r, 1)

            def chunk_body(c, _):
                pltpu.make_async_copy(
                    scores_hbm.at[row, pl.ds(0, csz)],
                    row_v.at[pl.ds(0, csz)], sem_in).wait()
                first = c == 0
                cbase = c * (tpc * NG)
                for v in range(NG // 16):
                    acc = jnp.where(first, negs16, gm_v[pl.ds(v * 16, 16)])
                    for t in range(tpc):
                        acc = jnp.maximum(
                            acc, row_v[pl.ds(cbase + t * NG + v * 16, 16)])
                    gm_v[pl.ds(v * 16, 16)] = acc
                return 0

            lax.fori_loop(0, NCHUNK, chunk_body, 0)

            m2 = negs16
            for j in range(16):
                x = jnp.maximum(gm_v[pl.ds(j * 32, 16)],
                                gm_v[pl.ds(j * 32 + 16, 16)])
                m2 = jnp.where(iota == j, jnp.max(x), m2)
            gm2_v[...] = m2

            def extract(kk, _):
                g2 = gm2_v[...]
                gmax = jnp.max(g2)
                jstar = jnp.min(jnp.where(g2 == gmax, iota, BIG))
                base = jstar * 32
                gva = gm_v[pl.ds(base, 16)]
                gvb = gm_v[pl.ds(base + 16, 16)]
                cand = jnp.minimum(
                    jnp.where(gva == gmax, base + iota, BIG),
                    jnp.where(gvb == gmax, base + 16 + iota, BIG))
                g = jnp.min(cand)

                def probe1(u, carry):
                    pv, m1v, m2v = carry
                    idx_u = g + NG * (u * 16 + iota)
                    val_u = plsc.load_gather(row_v, [idx_u])
                    pv = jnp.minimum(pv,
                                     jnp.where(val_u == gmax, idx_u, BIG))
                    m2v = jnp.maximum(m2v, jnp.minimum(val_u, m1v))
                    m1v = jnp.maximum(m1v, val_u)
                    return pv, m1v, m2v

                pvec, m1v, m2v = lax.fori_loop(
                    0, n // NG // 16, probe1,
                    (jnp.full((16,), BIG, jnp.int32), negs16, negs16))
                estar = jnp.min(pvec)
                estar_v = jnp.full((16,), estar, jnp.int32)
                elane = jnp.bitwise_and((estar - g) // NG, 15)
                nm = jnp.max(jnp.where(iota == elane, m2v, m1v))

                kk_v = jnp.full((16,), 0, jnp.int32) + kk
                plsc.store_scatter(row_v, [estar_v], negs16, mask=lane0)
                lane = g - base
                ia = jnp.where(lane < 16, lane, 99)
                ib = jnp.where(lane < 16, 99, lane - 16)
                gnew_a = jnp.where(iota == ia, nm, gva)
                gnew_b = jnp.where(iota == ib, nm, gvb)
                gm_v[pl.ds(base, 16)] = gnew_a
                gm_v[pl.ds(base + 16, 16)] = gnew_b
                pmax = jnp.max(jnp.maximum(gnew_a, gnew_b))
                plsc.store_scatter(gm2_v, [jnp.full((16,), jstar, jnp.int32)],
                                   jnp.full((16,), pmax, jnp.float32),
                                   mask=lane0)
                plsc.store_scatter(tv_v, [kk_v],
                                   jnp.full((16,), gmax, jnp.float32),
                                   mask=lane0)
                plsc.store_scatter(ti_v, [kk_v], estar_v, mask=lane0)
                return 0

            lax.fori_loop(0, K_TOP, extract, 0)

            tokcp = pltpu.async_copy(
                dbv_hbm.at[ti_v], tok_v.at[pl.ds(slot * K_TOP, K_TOP)],
                sem_tok)

            @pl.when(r + 1 < rows_per_w)
            def _():
                issue_row(row + 1)

            @pl.when(r >= 1)
            def _():
                pltpu.make_async_copy(out_hbm.at[row], ebd_v, sem_out).wait()

            sprev = 1 - slot
            old0 = tok_v[pl.ds(sprev * K_TOP, 16)]
            old1 = tok_v[pl.ds(sprev * K_TOP + 16, 16)]
            plsc.store_scatter(ebd_v, [old0], zeros16)
            plsc.store_scatter(ebd_v, [old1], zeros16)

            tv0 = tv_v[pl.ds(0, 16)]
            tv1 = tv_v[pl.ds(16, 16)]
            mx = jnp.max(jnp.maximum(tv0, tv1))
            e0 = jnp.exp((tv0 - mx) / BW)
            e1 = jnp.exp((tv1 - mx) / BW)
            scale = MIX / (zeros16 + jnp.sum(e0 + e1))
            w0 = e0 * scale
            w1 = e1 * scale

            tokcp.wait()
            t0 = tok_v[pl.ds(slot * K_TOP, 16)]
            t1 = tok_v[pl.ds(slot * K_TOP + 16, 16)]

            for j in range(16):
                mj = iota == j
                plsc.addupdate_scatter(ebd_v, [t0], w0, mask=mj)
                plsc.addupdate_scatter(ebd_v, [t1], w1, mask=mj)

            pltpu.async_copy(ebd_v, out_hbm.at[row], sem_out)
            return 0

        lax.fori_loop(0, rows_per_w, do_row, 0)

        pltpu.make_async_copy(out_hbm.at[0], ebd_v, sem_out).wait()

    return body(scores, db_values)


def _mix_body(lg_ref, ebd_ref, out_ref):
    lg = lg_ref[...]
    m = jnp.max(lg, axis=-1, keepdims=True)
    e = jnp.exp(lg - m)
    p = e / jnp.sum(e, axis=-1, keepdims=True)
    out_ref[...] = jnp.log((1.0 - MIX) * p + ebd_ref[...])


def _mix(lg, ebd, br):
    q, v = lg.shape
    return pl.pallas_call(
        _mix_body,
        grid=(q // br,),
        in_specs=[
            pl.BlockSpec((br, v), lambda i: (i, 0)),
            pl.BlockSpec((br, v), lambda i: (i, 0)),
        ],
        out_specs=pl.BlockSpec((br, v), lambda i: (i, 0)),
        out_shape=jax.ShapeDtypeStruct((q, v), jnp.float32),
    )(lg, ebd)


def kernel(hidden, logits, db_keys, db_values):
    b, s_len, d = hidden.shape
    vocab = logits.shape[-1]
    q = b * s_len
    h = hidden.reshape(q, d)
    lg = logits.reshape(q, vocab)

    scores = _scores(h, db_keys, 2048)
    ebd = _sc_midsection(scores, db_values.astype(jnp.int32), vocab)
    out = _mix(lg, ebd, 16)
    return out.reshape(b, s_len, vocab)


# speedup vs baseline: 1.1374x; 1.0197x over previous
import functools

import jax
import jax.numpy as jnp
from jax import lax
from jax.experimental import pallas as pl
from jax.experimental.pallas import tpu as pltpu
from jax.experimental.pallas import tpu_sc as plsc

K_TOP = 32
MIX = 0.25
BW = 10.0
NEG = -3.0e38
BIG = 2**30
NG = 512
NCHUNK = 16


def _scores_body(h_ref, k_ref, out_ref):
    kb = k_ref[...]
    s = lax.dot_general(h_ref[...], kb, (((1,), (1,)), ((), ())),
                        preferred_element_type=jnp.float32)
    ksq = jnp.sum(kb * kb, axis=1)
    out_ref[...] = 2.0 * s - ksq[None, :]


def _scores(h, db_keys, bn):
    q, d = h.shape
    n = db_keys.shape[0]
    return pl.pallas_call(
        _scores_body,
        grid=(n // bn,),
        in_specs=[
            pl.BlockSpec((q, d), lambda j: (0, 0)),
            pl.BlockSpec((bn, d), lambda j: (j, 0)),
        ],
        out_specs=pl.BlockSpec((q, bn), lambda j: (0, j)),
        out_shape=jax.ShapeDtypeStruct((q, n), jnp.float32),
    )(h, db_keys)


def _sc_midsection(scores, db_values, vocab):
    q, n = scores.shape
    info = plsc.get_sparse_core_info()
    nc, ns = info.num_cores, info.num_subcores
    nw = nc * ns
    rows_per_w = q // nw
    csz = n // NCHUNK
    tpc = (n // NG) // NCHUNK
    mesh = plsc.VectorSubcoreMesh(core_axis_name="c", subcore_axis_name="s")

    @functools.partial(
        pl.kernel,
        mesh=mesh,
        compiler_params=pltpu.CompilerParams(needs_layout_passes=False),
        out_type=jax.ShapeDtypeStruct((q, vocab), jnp.float32),
        scratch_types=[
            pltpu.VMEM((n,), jnp.float32),
            pltpu.VMEM((NG,), jnp.float32),
            pltpu.VMEM((16,), jnp.float32),
            pltpu.VMEM((K_TOP,), jnp.float32),
            pltpu.VMEM((K_TOP,), jnp.int32),
            pltpu.VMEM((2 * K_TOP,), jnp.int32),
            pltpu.VMEM((vocab,), jnp.float32),
            pltpu.SemaphoreType.DMA,
            pltpu.SemaphoreType.DMA,
            pltpu.SemaphoreType.DMA,
        ],
    )
    def body(scores_hbm, dbv_hbm, out_hbm, row_v, gm_v, gm2_v, tv_v, ti_v,
             tok_v, ebd_v, sem_in, sem_tok, sem_out):
        wid = lax.axis_index("s") * nc + lax.axis_index("c")
        iota = lax.iota(jnp.int32, 16)
        lane0 = iota == 0
        zeros16 = jnp.zeros((16,), jnp.float32)
        negs16 = jnp.full((16,), NEG, jnp.float32)

        def zero_body(i, _):
            ebd_v[pl.ds(i * 16, 16)] = zeros16
            return 0

        lax.fori_loop(0, vocab // 16, zero_body, 0)
        tok_v[pl.ds(0, 16)] = iota * 0
        tok_v[pl.ds(16, 16)] = iota * 0
        tok_v[pl.ds(32, 16)] = iota * 0
        tok_v[pl.ds(48, 16)] = iota * 0

        def issue_row(row):
            def issue(c, _):
                pltpu.async_copy(
                    scores_hbm.at[row, pl.ds(c * csz, csz)],
                    row_v.at[pl.ds(c * csz, csz)], sem_in)
                return 0

            lax.fori_loop(0, NCHUNK, issue, 0)

        issue_row(wid * rows_per_w)

        def do_row(r, _):
            row = wid * rows_per_w + r
            slot = jnp.bitwise_and(r, 1)

            def chunk_body(c, _):
                pltpu.make_async_copy(
                    scores_hbm.at[row, pl.ds(0, csz)],
                    row_v.at[pl.ds(0, csz)], sem_in).wait()
                first = c == 0
                cbase = c * (tpc * NG)
                for v in range(NG // 16):
                    acc = jnp.where(first, negs16, gm_v[pl.ds(v * 16, 16)])
                    for t in range(tpc):
                        acc = jnp.maximum(
                            acc, row_v[pl.ds(cbase + t * NG + v * 16, 16)])
                    gm_v[pl.ds(v * 16, 16)] = acc
                return 0

            lax.fori_loop(0, NCHUNK, chunk_body, 0)

            m2 = negs16
            for j in range(16):
                x = jnp.maximum(gm_v[pl.ds(j * 32, 16)],
                                gm_v[pl.ds(j * 32 + 16, 16)])
                m2 = jnp.where(iota == j, jnp.max(x), m2)
            gm2_v[...] = m2

            def extract(kk, _):
                g2 = gm2_v[...]
                gmax = jnp.max(g2)
                jstar = jnp.min(jnp.where(g2 == gmax, iota, BIG))
                base = jstar * 32
                gva = gm_v[pl.ds(base, 16)]
                gvb = gm_v[pl.ds(base + 16, 16)]
                cand = jnp.minimum(
                    jnp.where(gva == gmax, base + iota, BIG),
                    jnp.where(gvb == gmax, base + 16 + iota, BIG))
                g = jnp.min(cand)

                def probe1(u, carry):
                    pv, m1v, m2v = carry
                    idx_u = g + NG * (u * 16 + iota)
                    val_u = plsc.load_gather(row_v, [idx_u])
                    pv = jnp.minimum(pv,
                                     jnp.where(val_u == gmax, idx_u, BIG))
                    m2v = jnp.maximum(m2v, jnp.minimum(val_u, m1v))
                    m1v = jnp.maximum(m1v, val_u)
                    return pv, m1v, m2v

                pvec, m1v, m2v = lax.fori_loop(
                    0, n // NG // 16, probe1,
                    (jnp.full((16,), BIG, jnp.int32), negs16, negs16))
                estar = jnp.min(pvec)
                estar_v = jnp.full((16,), estar, jnp.int32)
                elane = jnp.bitwise_and((estar - g) // NG, 15)
                nm = jnp.max(jnp.where(iota == elane, m2v, m1v))

                kk_v = jnp.full((16,), 0, jnp.int32) + kk
                plsc.store_scatter(row_v, [estar_v], negs16, mask=lane0)
                lane = g - base
                ia = jnp.where(lane < 16, lane, 99)
                ib = jnp.where(lane < 16, 99, lane - 16)
                gnew_a = jnp.where(iota == ia, nm, gva)
                gnew_b = jnp.where(iota == ib, nm, gvb)
                gm_v[pl.ds(base, 16)] = gnew_a
                gm_v[pl.ds(base + 16, 16)] = gnew_b
                pmax = jnp.max(jnp.maximum(gnew_a, gnew_b))
                plsc.store_scatter(gm2_v, [jnp.full((16,), jstar, jnp.int32)],
                                   jnp.full((16,), pmax, jnp.float32),
                                   mask=lane0)
                plsc.store_scatter(tv_v, [kk_v],
                                   jnp.full((16,), gmax, jnp.float32),
                                   mask=lane0)
                plsc.store_scatter(ti_v, [kk_v], estar_v, mask=lane0)
                return 0

            lax.fori_loop(0, K_TOP, extract, 0)

            tokcp = pltpu.async_copy(
                dbv_hbm.at[ti_v], tok_v.at[pl.ds(slot * K_TOP, K_TOP)],
                sem_tok)

            @pl.when(r + 1 < rows_per_w)
            def _():
                issue_row(row + 1)

            @pl.when(r >= 1)
            def _():
                pltpu.make_async_copy(out_hbm.at[row], ebd_v, sem_out).wait()

            sprev = 1 - slot
            old0 = tok_v[pl.ds(sprev * K_TOP, 16)]
            old1 = tok_v[pl.ds(sprev * K_TOP + 16, 16)]
            plsc.store_scatter(ebd_v, [old0], zeros16)
            plsc.store_scatter(ebd_v, [old1], zeros16)

            tv0 = tv_v[pl.ds(0, 16)]
            tv1 = tv_v[pl.ds(16, 16)]
            mx = jnp.max(jnp.maximum(tv0, tv1))
            e0 = jnp.exp((tv0 - mx) / BW)
            e1 = jnp.exp((tv1 - mx) / BW)
            scale = MIX / (zeros16 + jnp.sum(e0 + e1))
            w0 = e0 * scale
            w1 = e1 * scale

            tokcp.wait()
            t0 = tok_v[pl.ds(slot * K_TOP, 16)]
            t1 = tok_v[pl.ds(slot * K_TOP + 16, 16)]

            for j in range(16):
                mj = iota == j
                plsc.addupdate_scatter(ebd_v, [t0], w0, mask=mj)
                plsc.addupdate_scatter(ebd_v, [t1], w1, mask=mj)

            pltpu.async_copy(ebd_v, out_hbm.at[row], sem_out)
            return 0

        lax.fori_loop(0, rows_per_w, do_row, 0)

        pltpu.make_async_copy(out_hbm.at[0], ebd_v, sem_out).wait()

    return body(scores, db_values)


def _mix_body(lg_ref, ebd_ref, out_ref):
    lg = lg_ref[...]
    m = jnp.max(lg, axis=-1, keepdims=True)
    e = jnp.exp(lg - m)
    p = e / jnp.sum(e, axis=-1, keepdims=True)
    out_ref[...] = jnp.log((1.0 - MIX) * p + ebd_ref[...])


def _mix(lg, ebd, br):
    q, v = lg.shape
    return pl.pallas_call(
        _mix_body,
        grid=(q // br,),
        in_specs=[
            pl.BlockSpec((br, v), lambda i: (i, 0)),
            pl.BlockSpec((br, v), lambda i: (i, 0)),
        ],
        out_specs=pl.BlockSpec((br, v), lambda i: (i, 0)),
        out_shape=jax.ShapeDtypeStruct((q, v), jnp.float32),
    )(lg, ebd)


def kernel(hidden, logits, db_keys, db_values):
    b, s_len, d = hidden.shape
    vocab = logits.shape[-1]
    q = b * s_len
    h = hidden.reshape(q, d)
    lg = logits.reshape(q, vocab)

    scores = _scores(h, db_keys, 2048)
    ebd = _sc_midsection(scores, db_values.astype(jnp.int32), vocab)
    out = _mix(lg, ebd, 32)
    return out.reshape(b, s_len, vocab)
